# baseline (speedup 1.0000x reference)
import functools

import jax
import jax.numpy as jnp
from jax import lax
from jax.experimental import pallas as pl
from jax.experimental.pallas import tpu as pltpu
from jax.experimental.pallas import tpu_sc as plsc

N = 10000
E = 320000
D = 128
K = 8
DD = 16
UNUM = 6000
B = 4096
NUM_LAYERS = 5
ROUTIT = 7

NW = 32
NTILE = 320
NPAD = NTILE * NW
NROWS = NTILE + 8
TRASH = NTILE
SCHUNK = 1024
GCHUNK = 128
NGI = SCHUNK // GCHUNK
EPAD2 = E + SCHUNK
GPW = 3 * B // NW

_GATHER_DNUMS = lax.GatherDimensionNumbers(
    offset_dims=(), collapsed_slice_dims=(0,), start_index_map=(0,))
_SWAP = [l ^ 8 for l in range(16)]


def _rsqrt16(x):
    i = lax.bitcast_convert_type(x, jnp.int32)
    i = jnp.int32(0x5F3759DF) - (i >> 1)
    y = lax.bitcast_convert_type(i, jnp.float32)
    for _ in range(3):
        y = y * (1.5 - 0.5 * x * y * y)
    return y


def _vswap(v):
    idx = (lax.iota(jnp.int32, 16) ^ 8).reshape(16, 1)
    return lax.gather(v, idx, _GATHER_DNUMS, (1,),
                      mode=lax.GatherScatterMode.PROMISE_IN_BOUNDS)


def _sread(ref, i):
    return ref[pl.ds(i, 16)][0]


def _perm_mats():
    idx_in = jnp.arange(D)
    col_out = (idx_in % DD) * K + idx_in // DD
    P = jnp.zeros((D, D), jnp.float32).at[idx_in, col_out].set(1.0)
    return P, P.T


def _init_body(x_ref, wt_ref, b_ref, g_ref, p_ref, o_ref):
    y = jnp.dot(x_ref[...], wt_ref[...], preferred_element_type=jnp.float32)
    y = jnp.maximum(y + b_ref[...], 0.0)
    ss = jnp.dot(y * y, g_ref[...], preferred_element_type=jnp.float32)
    yn = y / jnp.maximum(jnp.sqrt(ss), 1e-12)
    o_ref[...] = jnp.dot(yn, p_ref[...], preferred_element_type=jnp.float32)


def _init_layer(X, W, b, P):
    G = jnp.kron(jnp.eye(K, dtype=jnp.float32), jnp.ones((DD, DD), jnp.float32))
    return pl.pallas_call(
        _init_body,
        grid=(25,),
        in_specs=[
            pl.BlockSpec((400, D), lambda i: (i, 0)),
            pl.BlockSpec((D, D), lambda i: (0, 0)),
            pl.BlockSpec((1, D), lambda i: (0, 0)),
            pl.BlockSpec((D, D), lambda i: (0, 0)),
            pl.BlockSpec((D, D), lambda i: (0, 0)),
        ],
        out_specs=pl.BlockSpec((400, D), lambda i: (i, 0)),
        out_shape=jax.ShapeDtypeStruct((N, D), jnp.float32),
    )(X, W.T, b.reshape(1, D), G, P)


def _unperm_body(x_ref, pt_ref, o_ref):
    o_ref[...] = jnp.dot(x_ref[...], pt_ref[...],
                         preferred_element_type=jnp.float32)


def _unperm(rows, PT):
    return pl.pallas_call(
        _unperm_body,
        grid=(24,),
        in_specs=[
            pl.BlockSpec((512, D), lambda i: (i, 0)),
            pl.BlockSpec((D, D), lambda i: (0, 0)),
        ],
        out_specs=pl.BlockSpec((512, D), lambda i: (i, 0)),
        out_shape=jax.ShapeDtypeStruct((3 * B, D), jnp.float32),
    )(rows, PT)


def _routing_body(z_hbm, trg_hbm, srcl_hbm, est_hbm, zout_hbm,
                  c_t, acc_t, zt0, zt1, trg_v, src_v, est_v, sem0, sem1):
    wid = lax.axis_index("s") * 2 + lax.axis_index("c")
    n0 = wid * NTILE
    pltpu.sync_copy(est_hbm, est_v)
    e0 = _sread(est_v, wid)
    e1 = _sread(est_v, wid + 1)
    cb0 = (e0 // SCHUNK) * SCHUNK
    nbig = (e1 - cb0 + SCHUNK - 1) // SCHUNK
    pltpu.sync_copy(z_hbm.at[pl.ds(n0, NTILE)], c_t.at[pl.ds(0, NTILE)])
    zero16 = jnp.zeros((16,), jnp.float32)
    for r in range(NROWS - NTILE):
        for j in range(K):
            c_t[NTILE + r, pl.ds(DD * j, DD)] = zero16

    def desc(gi):
        ztb = (zt0, zt1)[gi % 2]
        sem = (sem0, sem1)[gi % 2]
        return pltpu.make_async_copy(z_hbm.at[trg_v.at[gi]], ztb, sem)

    def one_iter(t, carry):
        pltpu.sync_copy(z_hbm.at[pl.ds(n0, NTILE)], acc_t.at[pl.ds(0, NTILE)])

        def big_body(bi, c2):
            bb = cb0 + bi * SCHUNK
            br = pl.multiple_of(bb // GCHUNK, 8)
            pltpu.sync_copy(trg_hbm.at[pl.ds(br, NGI)], trg_v)
            pltpu.sync_copy(srcl_hbm.at[pl.ds(bb, SCHUNK)],
                            src_v.at[pl.ds(0, SCHUNK)])

            @pl.when(bb < e1)
            def _():
                desc(0).start()

            for gi in range(NGI):
                lo_g = bb + gi * GCHUNK
                if gi + 1 < NGI:
                    @pl.when(lo_g + GCHUNK < e1)
                    def _(gi=gi):
                        desc(gi + 1).start()

                @pl.when(lo_g < e1)
                def _(gi=gi, lo_g=lo_g):
                    desc(gi).wait()
                    ztb = (zt0, zt1)[gi % 2]

                    def _edges(g, cc):
                        ss, zts, wvs = [], [], []
                        for u in range(4):
                            il = g * 4 + u
                            sv = _sread(src_v, gi * GCHUNK + il)
                            ge = lo_g + il
                            ok = jnp.logical_and(ge >= e0, ge < e1)
                            ss.append(jnp.where(ok, sv, TRASH))
                        for u in range(4):
                            il = g * 4 + u
                            s = ss[u]
                            zt = [ztb[il, pl.ds(DD * j, DD)]
                                  for j in range(K)]
                            pr = [zt[j] * c_t[s, pl.ds(DD * j, DD)]
                                  for j in range(K)]
                            ps = (((pr[0] + pr[1]) + (pr[2] + pr[3]))
                                  + ((pr[4] + pr[5]) + (pr[6] + pr[7])))
                            zts.append(zt)
                            pp = ps + _vswap(ps)
                            ex = jnp.exp(pp)
                            wvs.append((ex + ex) / jnp.sum(ex))
                        for u in range(4):
                            for j in range(K):
                                plsc.addupdate(
                                    acc_t.at[ss[u], pl.ds(DD * j, DD)],
                                    wvs[u] * zts[u][j])
                        return cc

                    lax.fori_loop(0, GCHUNK // 4, _edges, 0)
            return c2

        lax.fori_loop(0, nbig, big_body, 0)

        def _norm(n, cc):
            vs = [acc_t[n, pl.ds(DD * j, DD)] for j in range(K)]
            qq = [v * v for v in vs]
            sq = (((qq[0] + qq[1]) + (qq[2] + qq[3]))
                  + ((qq[4] + qq[5]) + (qq[6] + qq[7])))
            ssp = sq + _vswap(sq)
            y = _rsqrt16(jnp.maximum(ssp, 1e-24))
            for j in range(K):
                c_t[n, pl.ds(DD * j, DD)] = vs[j] * y
            return cc

        lax.fori_loop(0, NTILE, _norm, 0)
        return carry

    lax.fori_loop(0, ROUTIT, one_iter, 0)

    def _relu(n, cc):
        for j in range(K):
            acc_t[n, pl.ds(DD * j, DD)] = jnp.maximum(
                c_t[n, pl.ds(DD * j, DD)], 0.0)
        return cc

    lax.fori_loop(0, NTILE, _relu, 0)
    pltpu.sync_copy(acc_t.at[pl.ds(0, NTILE)], zout_hbm.at[pl.ds(n0, NTILE)])


def _routing_layer(z, trg_p, srcl_p, est_p):
    mesh = plsc.VectorSubcoreMesh(core_axis_name="c", subcore_axis_name="s")
    f = pl.kernel(
        _routing_body,
        out_type=jax.ShapeDtypeStruct((NPAD, D), jnp.float32),
        mesh=mesh,
        scratch_types=[
            pltpu.VMEM((NROWS, D), jnp.float32),
            pltpu.VMEM((NROWS, D), jnp.float32),
            pltpu.VMEM((GCHUNK, D), jnp.float32),
            pltpu.VMEM((GCHUNK, D), jnp.float32),
            pltpu.VMEM((NGI, GCHUNK), jnp.int32),
            pltpu.VMEM((SCHUNK + 16,), jnp.int32),
            pltpu.VMEM((64,), jnp.int32),
            pltpu.SemaphoreType.DMA,
            pltpu.SemaphoreType.DMA,
        ],
        compiler_params=pltpu.CompilerParams(needs_layout_passes=False),
    )
    return f(z, trg_p, srcl_p, est_p)


def _gather_body(z_hbm, idx_hbm, out_hbm, idx_v, rows_v, sem):
    wid = lax.axis_index("s") * 2 + lax.axis_index("c")
    base = wid * GPW
    pltpu.sync_copy(idx_hbm.at[pl.ds(base, GPW)], idx_v)
    pltpu.async_copy(z_hbm.at[idx_v], rows_v, sem).wait()
    pltpu.sync_copy(rows_v, out_hbm.at[pl.ds(base, GPW)])


def _gather_rows(z, idx):
    mesh = plsc.VectorSubcoreMesh(core_axis_name="c", subcore_axis_name="s")
    f = pl.kernel(
        _gather_body,
        out_type=jax.ShapeDtypeStruct((3 * B, D), jnp.float32),
        mesh=mesh,
        scratch_types=[
            pltpu.VMEM((GPW,), jnp.int32),
            pltpu.VMEM((GPW, D), jnp.float32),
            pltpu.SemaphoreType.DMA,
        ],
    )
    return f(z, idx)


def kernel(X, edges, users, pos, neg, W, b):
    P, PT = _perm_mats()
    z0 = _init_layer(X, W, b, P)
    z = jnp.pad(z0, ((0, NPAD - N), (0, 0)))

    src = edges[0].astype(jnp.int32)
    trg = edges[1].astype(jnp.int32)
    order = jnp.argsort(src)
    src_s = src[order]
    trg_s = trg[order]
    est = jnp.searchsorted(
        src_s, (jnp.arange(1, NW, dtype=jnp.int32) * NTILE)).astype(jnp.int32)
    est_p = jnp.concatenate([
        jnp.zeros((1,), jnp.int32), est,
        jnp.full((1,), E, jnp.int32),
        jnp.zeros((64 - NW - 1,), jnp.int32)])
    srcl_p = jnp.pad(src_s - (src_s // NTILE) * NTILE, (0, SCHUNK))
    trg_p = jnp.pad(trg_s, (0, SCHUNK)).reshape(-1, GCHUNK)

    for _ in range(NUM_LAYERS):
        z = _routing_layer(z, trg_p, srcl_p, est_p)

    idx = jnp.concatenate(
        [users, pos + UNUM, neg + UNUM]).astype(jnp.int32)
    rows = _gather_rows(z, idx)
    out = _unperm(rows, PT)
    return out[:B], out[B:2 * B], out[2 * B:]


# speedup vs baseline: 1.2297x; 1.2297x over previous
import functools

import jax
import jax.numpy as jnp
from jax import lax
from jax.experimental import pallas as pl
from jax.experimental.pallas import tpu as pltpu
from jax.experimental.pallas import tpu_sc as plsc

N = 10000
E = 320000
D = 128
K = 8
DD = 16
UNUM = 6000
B = 4096
NUM_LAYERS = 5
ROUTIT = 7

NW = 32
NTILE = 320
NPAD = NTILE * NW
NROWS = NTILE + 8
TRASH = NTILE
SCHUNK = 1024
GCHUNK = 128
NGI = SCHUNK // GCHUNK
EPAD2 = E + SCHUNK
GPW = 3 * B // NW

_GATHER_DNUMS = lax.GatherDimensionNumbers(
    offset_dims=(), collapsed_slice_dims=(0,), start_index_map=(0,))
_SWAP = [l ^ 8 for l in range(16)]


def _rsqrt16(x):
    i = lax.bitcast_convert_type(x, jnp.int32)
    i = jnp.int32(0x5F3759DF) - (i >> 1)
    y = lax.bitcast_convert_type(i, jnp.float32)
    for _ in range(3):
        y = y * (1.5 - 0.5 * x * y * y)
    return y


def _vswap(v):
    idx = (lax.iota(jnp.int32, 16) ^ 8).reshape(16, 1)
    return lax.gather(v, idx, _GATHER_DNUMS, (1,),
                      mode=lax.GatherScatterMode.PROMISE_IN_BOUNDS)


def _sread(ref, i):
    return ref[pl.ds(i, 16)][0]


def _perm_mats():
    idx_in = jnp.arange(D)
    col_out = (idx_in % DD) * K + idx_in // DD
    P = jnp.zeros((D, D), jnp.float32).at[idx_in, col_out].set(1.0)
    return P, P.T


def _init_body(x_ref, wt_ref, b_ref, g_ref, p_ref, o_ref):
    y = jnp.dot(x_ref[...], wt_ref[...], preferred_element_type=jnp.float32)
    y = jnp.maximum(y + b_ref[...], 0.0)
    ss = jnp.dot(y * y, g_ref[...], preferred_element_type=jnp.float32)
    yn = y / jnp.maximum(jnp.sqrt(ss), 1e-12)
    o_ref[...] = jnp.dot(yn, p_ref[...], preferred_element_type=jnp.float32)


def _init_layer(X, W, b, P):
    G = jnp.kron(jnp.eye(K, dtype=jnp.float32), jnp.ones((DD, DD), jnp.float32))
    return pl.pallas_call(
        _init_body,
        grid=(25,),
        in_specs=[
            pl.BlockSpec((400, D), lambda i: (i, 0)),
            pl.BlockSpec((D, D), lambda i: (0, 0)),
            pl.BlockSpec((1, D), lambda i: (0, 0)),
            pl.BlockSpec((D, D), lambda i: (0, 0)),
            pl.BlockSpec((D, D), lambda i: (0, 0)),
        ],
        out_specs=pl.BlockSpec((400, D), lambda i: (i, 0)),
        out_shape=jax.ShapeDtypeStruct((N, D), jnp.float32),
    )(X, W.T, b.reshape(1, D), G, P)


def _unperm_body(x_ref, pt_ref, o_ref):
    o_ref[...] = jnp.dot(x_ref[...], pt_ref[...],
                         preferred_element_type=jnp.float32)


def _unperm(rows, PT):
    return pl.pallas_call(
        _unperm_body,
        grid=(24,),
        in_specs=[
            pl.BlockSpec((512, D), lambda i: (i, 0)),
            pl.BlockSpec((D, D), lambda i: (0, 0)),
        ],
        out_specs=pl.BlockSpec((512, D), lambda i: (i, 0)),
        out_shape=jax.ShapeDtypeStruct((3 * B, D), jnp.float32),
    )(rows, PT)


def _routing_body(z_hbm, trg_hbm, srcl_hbm, est_hbm, zout_hbm,
                  c_t, acc_t, zt0, zt1, trg_v, src_v, est_v, sem0, sem1):
    wid = lax.axis_index("s") * 2 + lax.axis_index("c")
    n0 = wid * NTILE
    pltpu.sync_copy(est_hbm, est_v)
    e0 = _sread(est_v, wid)
    e1 = _sread(est_v, wid + 1)
    cb0 = (e0 // SCHUNK) * SCHUNK
    nbig = (e1 - cb0 + SCHUNK - 1) // SCHUNK
    pltpu.sync_copy(z_hbm.at[pl.ds(n0, NTILE)], c_t.at[pl.ds(0, NTILE)])
    zero16 = jnp.zeros((16,), jnp.float32)
    for r in range(NROWS - NTILE):
        for j in range(K):
            c_t[NTILE + r, pl.ds(DD * j, DD)] = zero16

    def desc(gi):
        ztb = (zt0, zt1)[gi % 2]
        sem = (sem0, sem1)[gi % 2]
        return pltpu.make_async_copy(z_hbm.at[trg_v.at[gi]], ztb, sem)

    def one_iter(t, carry):
        pltpu.sync_copy(z_hbm.at[pl.ds(n0, NTILE)], acc_t.at[pl.ds(0, NTILE)])

        def big_body(bi, c2):
            bb = cb0 + bi * SCHUNK
            br = pl.multiple_of(bb // GCHUNK, 8)
            pltpu.sync_copy(trg_hbm.at[pl.ds(br, NGI)], trg_v)
            pltpu.sync_copy(srcl_hbm.at[pl.ds(bb, SCHUNK)],
                            src_v.at[pl.ds(0, SCHUNK)])

            @pl.when(bb < e1)
            def _():
                desc(0).start()

            for gi in range(NGI):
                lo_g = bb + gi * GCHUNK
                if gi + 1 < NGI:
                    @pl.when(lo_g + GCHUNK < e1)
                    def _(gi=gi):
                        desc(gi + 1).start()

                @pl.when(lo_g < e1)
                def _(gi=gi, lo_g=lo_g):
                    desc(gi).wait()
                    ztb = (zt0, zt1)[gi % 2]

                    def _edges(g, cc):
                        ss, zts, wvs = [], [], []
                        for u in range(4):
                            il = g * 4 + u
                            sv = _sread(src_v, gi * GCHUNK + il)
                            ge = lo_g + il
                            ok = jnp.logical_and(ge >= e0, ge < e1)
                            ss.append(jnp.where(ok, sv, TRASH))
                        for u in range(4):
                            il = g * 4 + u
                            s = ss[u]
                            zt = [ztb[il, pl.ds(DD * j, DD)]
                                  for j in range(K)]
                            ps = zt[0] * c_t[s, pl.ds(0, DD)]
                            for j in range(1, K):
                                ps = ps + zt[j] * c_t[s, pl.ds(DD * j, DD)]
                            zts.append(zt)
                            pp = ps + _vswap(ps)
                            ex = jnp.exp(pp)
                            wvs.append((ex + ex) / jnp.sum(ex))
                        for u in range(4):
                            for j in range(K):
                                plsc.addupdate(
                                    acc_t.at[ss[u], pl.ds(DD * j, DD)],
                                    wvs[u] * zts[u][j])
                        return cc

                    lax.fori_loop(0, GCHUNK // 4, _edges, 0)
            return c2

        lax.fori_loop(0, nbig, big_body, 0)

        def _norm(n, cc):
            vs = [acc_t[n, pl.ds(DD * j, DD)] for j in range(K)]
            sq = vs[0] * vs[0]
            for j in range(1, K):
                sq = sq + vs[j] * vs[j]
            ssp = sq + _vswap(sq)
            y = _rsqrt16(jnp.maximum(ssp, 1e-24))
            for j in range(K):
                c_t[n, pl.ds(DD * j, DD)] = vs[j] * y
            return cc

        lax.fori_loop(0, NTILE, _norm, 0)
        return carry

    lax.fori_loop(0, ROUTIT, one_iter, 0)

    def _relu(n, cc):
        for j in range(K):
            acc_t[n, pl.ds(DD * j, DD)] = jnp.maximum(
                c_t[n, pl.ds(DD * j, DD)], 0.0)
        return cc

    lax.fori_loop(0, NTILE, _relu, 0)
    pltpu.sync_copy(acc_t.at[pl.ds(0, NTILE)], zout_hbm.at[pl.ds(n0, NTILE)])


def _routing_layer(z, trg_p, srcl_p, est_p):
    mesh = plsc.VectorSubcoreMesh(core_axis_name="c", subcore_axis_name="s")
    f = pl.kernel(
        _routing_body,
        out_type=jax.ShapeDtypeStruct((NPAD, D), jnp.float32),
        mesh=mesh,
        scratch_types=[
            pltpu.VMEM((NROWS, D), jnp.float32),
            pltpu.VMEM((NROWS, D), jnp.float32),
            pltpu.VMEM((GCHUNK, D), jnp.float32),
            pltpu.VMEM((GCHUNK, D), jnp.float32),
            pltpu.VMEM((NGI, GCHUNK), jnp.int32),
            pltpu.VMEM((SCHUNK + 16,), jnp.int32),
            pltpu.VMEM((64,), jnp.int32),
            pltpu.SemaphoreType.DMA,
            pltpu.SemaphoreType.DMA,
        ],
        compiler_params=pltpu.CompilerParams(needs_layout_passes=False),
    )
    return f(z, trg_p, srcl_p, est_p)


def _gather_body(z_hbm, idx_hbm, out_hbm, idx_v, rows_v, sem):
    wid = lax.axis_index("s") * 2 + lax.axis_index("c")
    base = wid * GPW
    pltpu.sync_copy(idx_hbm.at[pl.ds(base, GPW)], idx_v)
    pltpu.async_copy(z_hbm.at[idx_v], rows_v, sem).wait()
    pltpu.sync_copy(rows_v, out_hbm.at[pl.ds(base, GPW)])


def _gather_rows(z, idx):
    mesh = plsc.VectorSubcoreMesh(core_axis_name="c", subcore_axis_name="s")
    f = pl.kernel(
        _gather_body,
        out_type=jax.ShapeDtypeStruct((3 * B, D), jnp.float32),
        mesh=mesh,
        scratch_types=[
            pltpu.VMEM((GPW,), jnp.int32),
            pltpu.VMEM((GPW, D), jnp.float32),
            pltpu.SemaphoreType.DMA,
        ],
    )
    return f(z, idx)


def kernel(X, edges, users, pos, neg, W, b):
    P, PT = _perm_mats()
    z0 = _init_layer(X, W, b, P)
    z = jnp.pad(z0, ((0, NPAD - N), (0, 0)))

    src = edges[0].astype(jnp.int32)
    trg = edges[1].astype(jnp.int32)
    order = jnp.argsort(src)
    src_s = src[order]
    trg_s = trg[order]
    est = jnp.searchsorted(
        src_s, (jnp.arange(1, NW, dtype=jnp.int32) * NTILE)).astype(jnp.int32)
    est_p = jnp.concatenate([
        jnp.zeros((1,), jnp.int32), est,
        jnp.full((1,), E, jnp.int32),
        jnp.zeros((64 - NW - 1,), jnp.int32)])
    srcl_p = jnp.pad(src_s - (src_s // NTILE) * NTILE, (0, SCHUNK))
    trg_p = jnp.pad(trg_s, (0, SCHUNK)).reshape(-1, GCHUNK)

    for _ in range(NUM_LAYERS):
        z = _routing_layer(z, trg_p, srcl_p, est_p)

    idx = jnp.concatenate(
        [users, pos + UNUM, neg + UNUM]).astype(jnp.int32)
    rows = _gather_rows(z, idx)
    out = _unperm(rows, PT)
    return out[:B], out[B:2 * B], out[2 * B:]
